# baseline (speedup 1.0000x reference)
import functools

import jax
import jax.numpy as jnp
from jax.experimental import pallas as pl
from jax.experimental.pallas import tpu as pltpu

_TOPK = 8
_N_GROUPS = 8
_TOPK_GROUPS = 4
_ROUTE_SCALE = 2.5
_NEG = -1e30


def _gate_kernel(x_ref, w_ref, wout_ref, iout_ref, sbuf_ref):
    i = pl.program_id(0)
    nt = pl.num_programs(0) - 1
    tb = x_ref.shape[0]

    @pl.when(i < nt)
    def _scores():
        z = jax.lax.dot_general(
            w_ref[...], x_ref[...],
            dimension_numbers=(((1,), (1,)), ((), ())),
            preferred_element_type=jnp.float32)
        sbuf_ref[i % 2] = 1.0 / (1.0 + jnp.exp(-z))

    @pl.when(i > 0)
    def _route():
        s = sbuf_ref[(i - 1) % 2]

        g = s.reshape(_N_GROUPS, 8, tb)
        m1 = jnp.max(g, axis=1)
        eq = g == m1[:, None, :]
        cnt = jnp.sum(eq.astype(jnp.float32), axis=1)
        m2 = jnp.where(cnt >= 2.0, m1,
                       jnp.max(jnp.where(eq, _NEG, g), axis=1))
        gs = m1 + m2

        giota = jax.lax.broadcasted_iota(jnp.int32, (_N_GROUPS, tb), 0)
        selg = giota >= _N_GROUPS
        gw = gs
        for _ in range(_TOPK_GROUPS):
            gm = jnp.max(gw, axis=0, keepdims=True)
            bi = jnp.min(jnp.where(gw == gm, giota, _N_GROUPS),
                         axis=0, keepdims=True)
            hit = giota == bi
            selg = selg | hit
            gw = jnp.where(hit, _NEG, gw)
        sel = jnp.broadcast_to(selg[:, None, :], (_N_GROUPS, 8, tb))
        masked = jnp.where(sel.reshape(64, tb), s, _NEG)

        eio = jax.lax.broadcasted_iota(jnp.int32, (64, tb), 0)
        wsum = jnp.zeros((1, tb), jnp.float32)
        for r in range(_TOPK):
            m = jnp.max(masked, axis=0, keepdims=True)
            bi = jnp.min(jnp.where(masked == m, eio, 64),
                         axis=0, keepdims=True)
            wout_ref[pl.ds(r, 1), :] = m
            iout_ref[pl.ds(r, 1), :] = bi
            wsum = wsum + m
            masked = jnp.where(eio == bi, _NEG, masked)

        wout_ref[...] = wout_ref[...] * (_ROUTE_SCALE / wsum)


@functools.partial(jax.jit, static_argnames=())
def kernel(x, weight):
    t, d = x.shape
    e = weight.shape[0]
    tb = 512
    if t % tb != 0:
        tb = 256 if t % 256 == 0 else t
    nt = t // tb
    w8, i8 = pl.pallas_call(
        _gate_kernel,
        grid=(nt + 1,),
        in_specs=[
            pl.BlockSpec((tb, d), lambda i: (jnp.minimum(i, nt - 1), 0)),
            pl.BlockSpec((e, d), lambda i: (0, 0)),
        ],
        out_specs=[
            pl.BlockSpec((_TOPK, tb), lambda i: (0, jnp.maximum(i - 1, 0))),
            pl.BlockSpec((_TOPK, tb), lambda i: (0, jnp.maximum(i - 1, 0))),
        ],
        out_shape=[
            jax.ShapeDtypeStruct((_TOPK, t), jnp.float32),
            jax.ShapeDtypeStruct((_TOPK, t), jnp.int32),
        ],
        scratch_shapes=[pltpu.VMEM((2, e, tb), jnp.float32)],
        compiler_params=pltpu.CompilerParams(
            dimension_semantics=("arbitrary",)),
    )(x, weight)
    return w8.T.astype(x.dtype), i8.T


# speedup vs baseline: 1.1044x; 1.1044x over previous
import functools

import jax
import jax.numpy as jnp
from jax.experimental import pallas as pl
from jax.experimental.pallas import tpu as pltpu

_TOPK = 8
_N_GROUPS = 8
_TOPK_GROUPS = 4
_ROUTE_SCALE = 2.5
_NEG = -1e30


def _gate_kernel(x_ref, w_ref, wout_ref, iout_ref, sbuf_ref):
    tb = x_ref.shape[0]

    s = sbuf_ref[...]

    z = jax.lax.dot_general(
        w_ref[...], x_ref[...],
        dimension_numbers=(((1,), (1,)), ((), ())),
        preferred_element_type=jnp.float32)
    sbuf_ref[...] = 1.0 / (1.0 + jnp.exp(-z))

    g = s.reshape(_N_GROUPS, 8, tb)
    m1 = jnp.max(g, axis=1)
    eq = g == m1[:, None, :]
    cnt = jnp.sum(eq.astype(jnp.float32), axis=1)
    m2 = jnp.where(cnt >= 2.0, m1,
                   jnp.max(jnp.where(eq, _NEG, g), axis=1))
    gs = m1 + m2

    giota = jax.lax.broadcasted_iota(jnp.int32, (_N_GROUPS, tb), 0)
    selg = giota >= _N_GROUPS
    gw = gs
    for _ in range(_TOPK_GROUPS):
        gm = jnp.max(gw, axis=0, keepdims=True)
        bi = jnp.min(jnp.where(gw == gm, giota, _N_GROUPS),
                     axis=0, keepdims=True)
        hit = giota == bi
        selg = selg | hit
        gw = jnp.where(hit, _NEG, gw)
    sel = jnp.broadcast_to(selg[:, None, :], (_N_GROUPS, 8, tb))
    masked = jnp.where(sel.reshape(64, tb), s, _NEG)

    eio = jax.lax.broadcasted_iota(jnp.int32, (64, tb), 0)
    wsum = jnp.zeros((1, tb), jnp.float32)
    for r in range(_TOPK):
        m = jnp.max(masked, axis=0, keepdims=True)
        bi = jnp.min(jnp.where(masked == m, eio, 64),
                     axis=0, keepdims=True)
        wout_ref[pl.ds(r, 1), :] = m
        iout_ref[pl.ds(r, 1), :] = bi
        wsum = wsum + m
        masked = jnp.where(eio == bi, _NEG, masked)

    wout_ref[...] = wout_ref[...] * (_ROUTE_SCALE / wsum)


@functools.partial(jax.jit, static_argnames=())
def kernel(x, weight):
    t, d = x.shape
    e = weight.shape[0]
    tb = 512
    if t % tb != 0:
        tb = 256 if t % 256 == 0 else t
    nt = t // tb
    w8, i8 = pl.pallas_call(
        _gate_kernel,
        grid=(nt + 1,),
        in_specs=[
            pl.BlockSpec((tb, d), lambda i: (jnp.minimum(i, nt - 1), 0)),
            pl.BlockSpec((e, d), lambda i: (0, 0)),
        ],
        out_specs=[
            pl.BlockSpec((_TOPK, tb), lambda i: (0, jnp.maximum(i - 1, 0))),
            pl.BlockSpec((_TOPK, tb), lambda i: (0, jnp.maximum(i - 1, 0))),
        ],
        out_shape=[
            jax.ShapeDtypeStruct((_TOPK, t), jnp.float32),
            jax.ShapeDtypeStruct((_TOPK, t), jnp.int32),
        ],
        scratch_shapes=[pltpu.VMEM((e, tb), jnp.float32)],
        compiler_params=pltpu.CompilerParams(
            dimension_semantics=("arbitrary",)),
    )(x, weight)
    return w8.T.astype(x.dtype), i8.T


# speedup vs baseline: 1.5382x; 1.3928x over previous
import functools

import jax
import jax.numpy as jnp
from jax.experimental import pallas as pl
from jax.experimental.pallas import tpu as pltpu

_TOPK = 8
_N_GROUPS = 8
_TOPK_GROUPS = 4
_ROUTE_SCALE = 2.5
_NEG = -1e30
_CW = 512


def _route_chunk(s, c0, cw, wout_ref, iout_ref):
    g = s.reshape(_N_GROUPS, 8, cw)
    m1 = jnp.max(g, axis=1)
    eq = g == m1[:, None, :]
    cnt = jnp.sum(eq.astype(jnp.float32), axis=1)
    m2 = jnp.where(cnt >= 2.0, m1,
                   jnp.max(jnp.where(eq, _NEG, g), axis=1))
    gw = m1 + m2

    giota = jax.lax.broadcasted_iota(jnp.int32, (_N_GROUPS, cw), 0)
    selg = giota >= _N_GROUPS
    for _ in range(_TOPK_GROUPS):
        gm = jnp.max(gw, axis=0, keepdims=True)
        bi = jnp.min(jnp.where(gw == gm, giota, _N_GROUPS),
                     axis=0, keepdims=True)
        hit = giota == bi
        selg = selg | hit
        gw = jnp.where(hit, _NEG, gw)
    sel = jnp.broadcast_to(selg[:, None, :], (_N_GROUPS, 8, cw))
    masked = jnp.where(sel.reshape(64, cw), s, _NEG)

    eio = jax.lax.broadcasted_iota(jnp.int32, (64, cw), 0)
    wsum = jnp.zeros((1, cw), jnp.float32)
    for r in range(_TOPK):
        m = jnp.max(masked, axis=0, keepdims=True)
        bi = jnp.min(jnp.where(masked == m, eio, 64),
                     axis=0, keepdims=True)
        wout_ref[pl.ds(r, 1), pl.ds(c0, cw)] = m
        iout_ref[pl.ds(r, 1), pl.ds(c0, cw)] = bi
        wsum = wsum + m
        masked = jnp.where(eio == bi, _NEG, masked)

    wout_ref[:, pl.ds(c0, cw)] = (
        wout_ref[:, pl.ds(c0, cw)] * (_ROUTE_SCALE / wsum))


def _gate_kernel(x_ref, w_ref, wout_ref, iout_ref, sbuf_ref):
    tb = x_ref.shape[0]
    cw = min(_CW, tb)

    for c in range(tb // cw):
        _route_chunk(sbuf_ref[:, pl.ds(c * cw, cw)], c * cw, cw,
                     wout_ref, iout_ref)

    z = jax.lax.dot_general(
        w_ref[...], x_ref[...],
        dimension_numbers=(((1,), (1,)), ((), ())),
        preferred_element_type=jnp.float32)
    sbuf_ref[...] = 1.0 / (1.0 + jnp.exp(-z))


@functools.partial(jax.jit, static_argnames=())
def kernel(x, weight):
    t, d = x.shape
    e = weight.shape[0]
    tb = 2048
    while tb > 8 and t % tb != 0:
        tb //= 2
    nt = t // tb
    w8, i8 = pl.pallas_call(
        _gate_kernel,
        grid=(nt + 1,),
        in_specs=[
            pl.BlockSpec((tb, d), lambda i: (jnp.minimum(i, nt - 1), 0)),
            pl.BlockSpec((e, d), lambda i: (0, 0)),
        ],
        out_specs=[
            pl.BlockSpec((_TOPK, tb), lambda i: (0, jnp.maximum(i - 1, 0))),
            pl.BlockSpec((_TOPK, tb), lambda i: (0, jnp.maximum(i - 1, 0))),
        ],
        out_shape=[
            jax.ShapeDtypeStruct((_TOPK, t), jnp.float32),
            jax.ShapeDtypeStruct((_TOPK, t), jnp.int32),
        ],
        scratch_shapes=[pltpu.VMEM((e, tb), jnp.float32)],
        compiler_params=pltpu.CompilerParams(
            dimension_semantics=("arbitrary",)),
    )(x, weight)
    return w8.T.astype(x.dtype), i8.T
